# baseline (speedup 1.0000x reference)
import functools

import jax
import jax.numpy as jnp
from jax import lax
from jax.experimental import pallas as pl
from jax.experimental.pallas import tpu as pltpu
from jax.experimental.pallas import tpu_sc as plsc

B = 32
DIM = 768
K = 10
NUM_ANSWERS = 100000
RUNW = 16
BIGI = 2**30

R = 2000
NB = NUM_ANSWERS // R


def _dot_t(x, w):
    return lax.dot_general(x.astype(jnp.bfloat16), w.astype(jnp.bfloat16),
                           (((1,), (1,)), ((), ())),
                           preferred_element_type=jnp.float32)


def _fused_query(visual_feat, text_feat, in_proj_w, in_proj_b, out_proj_w,
                 out_proj_b, fusion_w, fusion_b, ln_g, ln_b, sim_w, sim_b):
    b = visual_feat.shape[0]
    vf = visual_feat[:, None, :]
    tf = text_feat[:, None, :]
    wq, wk, wv = jnp.split(in_proj_w, 3, axis=0)
    bq, bk, bv = jnp.split(in_proj_b, 3)
    q = tf @ wq.T + bq
    k = vf @ wk.T + bk
    v = vf @ wv.T + bv
    hd = DIM // 8
    q = q.reshape(b, 1, 8, hd).transpose(0, 2, 1, 3)
    k = k.reshape(b, 1, 8, hd).transpose(0, 2, 1, 3)
    v = v.reshape(b, 1, 8, hd).transpose(0, 2, 1, 3)
    attn = jax.nn.softmax(
        (q @ jnp.swapaxes(k, -1, -2)) / jnp.sqrt(jnp.float32(hd)), axis=-1)
    out = (attn @ v).transpose(0, 2, 1, 3).reshape(b, 1, DIM)
    attended = (out @ out_proj_w.T + out_proj_b)[:, 0, :]
    fused_in = jnp.concatenate([visual_feat, attended], axis=-1)
    fused = fused_in @ fusion_w.T + fusion_b
    mu = jnp.mean(fused, axis=-1, keepdims=True)
    var = jnp.mean((fused - mu) ** 2, axis=-1, keepdims=True)
    fused = (fused - mu) / jnp.sqrt(var + 1e-5) * ln_g + ln_b
    fused = jax.nn.gelu(fused, approximate=False)
    fused_proj = fused @ sim_w.T + sim_b
    n = jnp.linalg.norm(fused_proj, axis=-1, keepdims=True)
    return fused_proj / jnp.clip(n, 1e-12, None)


def _sim_body(fn_ref, a_ref, scores_ref, idx_ref, run_s, run_i):
    i = pl.program_id(0)

    @pl.when(i == 0)
    def _init():
        run_s[...] = jnp.full((B, RUNW), -jnp.inf, jnp.float32)
        run_i[...] = jnp.full((B, RUNW), BIGI, jnp.int32)

    a = a_ref[...]
    ones = jnp.ones((1, DIM), jnp.float32)
    ss = lax.dot_general(a * a, ones, (((1,), (1,)), ((), ())),
                         precision=lax.Precision.HIGHEST,
                         preferred_element_type=jnp.float32)
    an = a / jnp.maximum(jnp.sqrt(ss), 1e-12)
    s = _dot_t(fn_ref[...], an)

    iota = lax.broadcasted_iota(jnp.int32, (B, R), 1)
    bs_l, bi_l = [], []
    for _ in range(K):
        m = jnp.max(s, axis=1, keepdims=True)
        pos = jnp.min(jnp.where(s == m, iota, R), axis=1, keepdims=True)
        bs_l.append(m)
        bi_l.append(pos)
        s = jnp.where(iota == pos, -jnp.inf, s)
    bs = jnp.concatenate(bs_l, axis=1)
    bi = jnp.concatenate(bi_l, axis=1) + i * R

    cs = jnp.concatenate([run_s[...], bs], axis=1)
    ci = jnp.concatenate([run_i[...], bi], axis=1)
    os_l, oi_l = [], []
    for _ in range(K):
        m = jnp.max(cs, axis=1, keepdims=True)
        sel = jnp.min(jnp.where(cs == m, ci, BIGI), axis=1, keepdims=True)
        os_l.append(m)
        oi_l.append(sel)
        cs = jnp.where(ci == sel, -jnp.inf, cs)
    new_s = jnp.concatenate(
        os_l + [jnp.full((B, RUNW - K), -jnp.inf, jnp.float32)], axis=1)
    new_i = jnp.concatenate(
        oi_l + [jnp.full((B, RUNW - K), BIGI, jnp.int32)], axis=1)
    run_s[...] = new_s
    run_i[...] = new_i

    @pl.when(i == NB - 1)
    def _fin():
        scores_ref[...] = new_s[:, :K]
        idx_ref[...] = new_i[:, :K]


def _sim_topk(fused_norm, answer_embeddings):
    return pl.pallas_call(
        _sim_body,
        grid=(NB,),
        in_specs=[
            pl.BlockSpec((B, DIM), lambda i: (0, 0)),
            pl.BlockSpec((R, DIM), lambda i: (i, 0)),
        ],
        out_specs=[
            pl.BlockSpec((B, K), lambda i: (0, 0)),
            pl.BlockSpec((B, K), lambda i: (0, 0)),
        ],
        out_shape=[
            jax.ShapeDtypeStruct((B, K), jnp.float32),
            jax.ShapeDtypeStruct((B, K), jnp.int32),
        ],
        scratch_shapes=[
            pltpu.VMEM((B, RUNW), jnp.float32),
            pltpu.VMEM((B, RUNW), jnp.int32),
        ],
    )(fused_norm, answer_embeddings)


def _gather_sc(answer_embeddings, idx_pad):
    info = plsc.get_sparse_core_info()
    nc = info.num_cores

    mesh = plsc.VectorSubcoreMesh(core_axis_name="c", subcore_axis_name="s")

    @functools.partial(
        pl.kernel,
        out_type=jax.ShapeDtypeStruct((B, RUNW, DIM), jnp.float32),
        mesh=mesh,
        scratch_types=[
            pltpu.VMEM((RUNW,), jnp.int32),
            pltpu.VMEM((RUNW, DIM), jnp.float32),
            pltpu.SemaphoreType.DMA,
        ],
    )
    def gather_k(table_hbm, idx_hbm, out_hbm, idx_v, rows_v, sem):
        wid = lax.axis_index("s") * nc + lax.axis_index("c")
        pltpu.sync_copy(idx_hbm.at[wid], idx_v)
        pltpu.async_copy(table_hbm.at[idx_v], rows_v, sem).wait()
        pltpu.sync_copy(rows_v, out_hbm.at[wid])

    return gather_k(answer_embeddings, idx_pad)


def kernel(visual_feat, text_feat, answer_embeddings, in_proj_w, in_proj_b,
           out_proj_w, out_proj_b, fusion_w, fusion_b, ln_g, ln_b, sim_w,
           sim_b):
    fused_norm = _fused_query(visual_feat, text_feat, in_proj_w, in_proj_b,
                              out_proj_w, out_proj_b, fusion_w, fusion_b,
                              ln_g, ln_b, sim_w, sim_b)
    scores, idx = _sim_topk(fused_norm, answer_embeddings)
    idx_pad = jnp.concatenate(
        [idx, jnp.zeros((B, RUNW - K), jnp.int32)], axis=1)
    emb = _gather_sc(answer_embeddings, idx_pad)[:, :K, :]
    return scores, idx, emb


# speedup vs baseline: 1.4946x; 1.4946x over previous
import functools

import jax
import jax.numpy as jnp
from jax import lax
from jax.experimental import pallas as pl
from jax.experimental.pallas import tpu as pltpu
from jax.experimental.pallas import tpu_sc as plsc

B = 32
DIM = 768
K = 10
NUM_ANSWERS = 100000
RUNW = 16
BIGI = 2**30

R = 4000
NB = NUM_ANSWERS // R
Q = 4
RQ = R // Q


def _dot_t(x, w):
    return lax.dot_general(x.astype(jnp.bfloat16), w.astype(jnp.bfloat16),
                           (((1,), (1,)), ((), ())),
                           preferred_element_type=jnp.float32)


def _fused_query(visual_feat, text_feat, in_proj_w, in_proj_b, out_proj_w,
                 out_proj_b, fusion_w, fusion_b, ln_g, ln_b, sim_w, sim_b):
    b = visual_feat.shape[0]
    vf = visual_feat[:, None, :]
    tf = text_feat[:, None, :]
    wq, wk, wv = jnp.split(in_proj_w, 3, axis=0)
    bq, bk, bv = jnp.split(in_proj_b, 3)
    q = tf @ wq.T + bq
    k = vf @ wk.T + bk
    v = vf @ wv.T + bv
    hd = DIM // 8
    q = q.reshape(b, 1, 8, hd).transpose(0, 2, 1, 3)
    k = k.reshape(b, 1, 8, hd).transpose(0, 2, 1, 3)
    v = v.reshape(b, 1, 8, hd).transpose(0, 2, 1, 3)
    attn = jax.nn.softmax(
        (q @ jnp.swapaxes(k, -1, -2)) / jnp.sqrt(jnp.float32(hd)), axis=-1)
    out = (attn @ v).transpose(0, 2, 1, 3).reshape(b, 1, DIM)
    attended = (out @ out_proj_w.T + out_proj_b)[:, 0, :]
    fused_in = jnp.concatenate([visual_feat, attended], axis=-1)
    fused = fused_in @ fusion_w.T + fusion_b
    mu = jnp.mean(fused, axis=-1, keepdims=True)
    var = jnp.mean((fused - mu) ** 2, axis=-1, keepdims=True)
    fused = (fused - mu) / jnp.sqrt(var + 1e-5) * ln_g + ln_b
    fused = jax.nn.gelu(fused, approximate=False)
    fused_proj = fused @ sim_w.T + sim_b
    n = jnp.linalg.norm(fused_proj, axis=-1, keepdims=True)
    return fused_proj / jnp.clip(n, 1e-12, None)


def _sim_body(fn_ref, a_ref, scores_ref, idx_ref, run_s, run_i):
    i = pl.program_id(0)

    @pl.when(i == 0)
    def _init():
        run_s[...] = jnp.full((B, RUNW), -jnp.inf, jnp.float32)
        run_i[...] = jnp.full((B, RUNW), BIGI, jnp.int32)

    a = a_ref[...]
    ones = jnp.ones((1, DIM), jnp.float32)
    ss = lax.dot_general(a * a, ones, (((1,), (1,)), ((), ())),
                         precision=lax.Precision.HIGHEST,
                         preferred_element_type=jnp.float32)
    an = a / jnp.maximum(jnp.sqrt(ss), 1e-12)
    s = _dot_t(fn_ref[...], an)

    bs_l, bi_l = [], []
    iota = lax.broadcasted_iota(jnp.int32, (B, RQ), 1)
    for q in range(Q):
        sq_ = s[:, q * RQ:(q + 1) * RQ]
        for _ in range(K):
            m = jnp.max(sq_, axis=1, keepdims=True)
            pos = jnp.min(jnp.where(sq_ == m, iota, RQ), axis=1, keepdims=True)
            bs_l.append(m)
            bi_l.append(pos + q * RQ)
            sq_ = jnp.where(iota == pos, -jnp.inf, sq_)
    bs = jnp.concatenate(bs_l, axis=1)
    bi = jnp.concatenate(bi_l, axis=1) + i * R

    cs = jnp.concatenate([run_s[...], bs], axis=1)
    ci = jnp.concatenate([run_i[...], bi], axis=1)
    os_l, oi_l = [], []
    for _ in range(K):
        m = jnp.max(cs, axis=1, keepdims=True)
        sel = jnp.min(jnp.where(cs == m, ci, BIGI), axis=1, keepdims=True)
        os_l.append(m)
        oi_l.append(sel)
        cs = jnp.where(ci == sel, -jnp.inf, cs)
    new_s = jnp.concatenate(
        os_l + [jnp.full((B, RUNW - K), -jnp.inf, jnp.float32)], axis=1)
    new_i = jnp.concatenate(
        oi_l + [jnp.full((B, RUNW - K), BIGI, jnp.int32)], axis=1)
    run_s[...] = new_s
    run_i[...] = new_i

    @pl.when(i == NB - 1)
    def _fin():
        scores_ref[...] = new_s[:, :K]
        idx_ref[...] = new_i[:, :K]


def _sim_topk(fused_norm, answer_embeddings):
    return pl.pallas_call(
        _sim_body,
        grid=(NB,),
        in_specs=[
            pl.BlockSpec((B, DIM), lambda i: (0, 0)),
            pl.BlockSpec((R, DIM), lambda i: (i, 0)),
        ],
        out_specs=[
            pl.BlockSpec((B, K), lambda i: (0, 0)),
            pl.BlockSpec((B, K), lambda i: (0, 0)),
        ],
        out_shape=[
            jax.ShapeDtypeStruct((B, K), jnp.float32),
            jax.ShapeDtypeStruct((B, K), jnp.int32),
        ],
        scratch_shapes=[
            pltpu.VMEM((B, RUNW), jnp.float32),
            pltpu.VMEM((B, RUNW), jnp.int32),
        ],
    )(fused_norm, answer_embeddings)


def _gather_sc(answer_embeddings, idx_pad):
    info = plsc.get_sparse_core_info()
    nc = info.num_cores

    mesh = plsc.VectorSubcoreMesh(core_axis_name="c", subcore_axis_name="s")

    @functools.partial(
        pl.kernel,
        out_type=jax.ShapeDtypeStruct((B, RUNW, DIM), jnp.float32),
        mesh=mesh,
        scratch_types=[
            pltpu.VMEM((RUNW,), jnp.int32),
            pltpu.VMEM((RUNW, DIM), jnp.float32),
            pltpu.SemaphoreType.DMA,
        ],
    )
    def gather_k(table_hbm, idx_hbm, out_hbm, idx_v, rows_v, sem):
        wid = lax.axis_index("s") * nc + lax.axis_index("c")
        pltpu.sync_copy(idx_hbm.at[wid], idx_v)
        pltpu.async_copy(table_hbm.at[idx_v], rows_v, sem).wait()
        pltpu.sync_copy(rows_v, out_hbm.at[wid])

    return gather_k(answer_embeddings, idx_pad)


def kernel(visual_feat, text_feat, answer_embeddings, in_proj_w, in_proj_b,
           out_proj_w, out_proj_b, fusion_w, fusion_b, ln_g, ln_b, sim_w,
           sim_b):
    fused_norm = _fused_query(visual_feat, text_feat, in_proj_w, in_proj_b,
                              out_proj_w, out_proj_b, fusion_w, fusion_b,
                              ln_g, ln_b, sim_w, sim_b)
    scores, idx = _sim_topk(fused_norm, answer_embeddings)
    idx_pad = jnp.concatenate(
        [idx, jnp.zeros((B, RUNW - K), jnp.int32)], axis=1)
    emb = _gather_sc(answer_embeddings, idx_pad)[:, :K, :]
    return scores, idx, emb


# speedup vs baseline: 1.6888x; 1.1299x over previous
import functools

import jax
import jax.numpy as jnp
from jax import lax
from jax.experimental import pallas as pl
from jax.experimental.pallas import tpu as pltpu
from jax.experimental.pallas import tpu_sc as plsc

B = 32
DIM = 768
K = 10
NUM_ANSWERS = 100000
RUNW = 16
BIGI = 2**30

R = 5000
NB = NUM_ANSWERS // R
Q = 5
RQ = R // Q


def _dot_t(x, w):
    return lax.dot_general(x.astype(jnp.bfloat16), w.astype(jnp.bfloat16),
                           (((1,), (1,)), ((), ())),
                           preferred_element_type=jnp.float32)


def _fused_query(visual_feat, text_feat, in_proj_w, in_proj_b, out_proj_w,
                 out_proj_b, fusion_w, fusion_b, ln_g, ln_b, sim_w, sim_b):
    wv = in_proj_w[2 * DIM:]
    bv = in_proj_b[2 * DIM:]
    v = _dot_t(visual_feat, wv) + bv
    attended = _dot_t(v, out_proj_w) + out_proj_b
    fused_in = jnp.concatenate([visual_feat, attended], axis=-1)
    fused = _dot_t(fused_in, fusion_w) + fusion_b
    mu = jnp.mean(fused, axis=-1, keepdims=True)
    var = jnp.mean((fused - mu) ** 2, axis=-1, keepdims=True)
    fused = (fused - mu) / jnp.sqrt(var + 1e-5) * ln_g + ln_b
    fused = jax.nn.gelu(fused, approximate=False)
    fused_proj = _dot_t(fused, sim_w) + sim_b
    n = jnp.linalg.norm(fused_proj, axis=-1, keepdims=True)
    return fused_proj / jnp.clip(n, 1e-12, None)


def _sim_body(fn_ref, a_ref, scores_ref, idx_ref, run_s, run_i):
    i = pl.program_id(0)

    @pl.when(i == 0)
    def _init():
        run_s[...] = jnp.full((B, RUNW), -jnp.inf, jnp.float32)
        run_i[...] = jnp.full((B, RUNW), BIGI, jnp.int32)

    a = a_ref[...]
    ones = jnp.ones((1, DIM), jnp.float32)
    ss = lax.dot_general(a * a, ones, (((1,), (1,)), ((), ())),
                         precision=lax.Precision.HIGHEST,
                         preferred_element_type=jnp.float32)
    an = a / jnp.maximum(jnp.sqrt(ss), 1e-12)
    s = _dot_t(fn_ref[...], an)

    bs_l, bi_l = [], []
    iota = lax.broadcasted_iota(jnp.int32, (B, RQ), 1)
    for q in range(Q):
        sq_ = s[:, q * RQ:(q + 1) * RQ]
        for _ in range(K):
            m = jnp.max(sq_, axis=1, keepdims=True)
            pos = jnp.min(jnp.where(sq_ == m, iota, RQ), axis=1, keepdims=True)
            bs_l.append(m)
            bi_l.append(pos + q * RQ)
            sq_ = jnp.where(iota == pos, -jnp.inf, sq_)
    bs = jnp.concatenate(bs_l, axis=1)
    bi = jnp.concatenate(bi_l, axis=1) + i * R

    cs = jnp.concatenate([run_s[...], bs], axis=1)
    ci = jnp.concatenate([run_i[...], bi], axis=1)
    os_l, oi_l = [], []
    for _ in range(K):
        m = jnp.max(cs, axis=1, keepdims=True)
        sel = jnp.min(jnp.where(cs == m, ci, BIGI), axis=1, keepdims=True)
        os_l.append(m)
        oi_l.append(sel)
        cs = jnp.where(ci == sel, -jnp.inf, cs)
    new_s = jnp.concatenate(
        os_l + [jnp.full((B, RUNW - K), -jnp.inf, jnp.float32)], axis=1)
    new_i = jnp.concatenate(
        oi_l + [jnp.full((B, RUNW - K), BIGI, jnp.int32)], axis=1)
    run_s[...] = new_s
    run_i[...] = new_i

    @pl.when(i == NB - 1)
    def _fin():
        scores_ref[...] = new_s[:, :K]
        idx_ref[...] = new_i[:, :K]


def _sim_topk(fused_norm, answer_embeddings):
    return pl.pallas_call(
        _sim_body,
        grid=(NB,),
        in_specs=[
            pl.BlockSpec((B, DIM), lambda i: (0, 0)),
            pl.BlockSpec((R, DIM), lambda i: (i, 0)),
        ],
        out_specs=[
            pl.BlockSpec((B, K), lambda i: (0, 0)),
            pl.BlockSpec((B, K), lambda i: (0, 0)),
        ],
        out_shape=[
            jax.ShapeDtypeStruct((B, K), jnp.float32),
            jax.ShapeDtypeStruct((B, K), jnp.int32),
        ],
        scratch_shapes=[
            pltpu.VMEM((B, RUNW), jnp.float32),
            pltpu.VMEM((B, RUNW), jnp.int32),
        ],
    )(fused_norm, answer_embeddings)


def _gather_sc(answer_embeddings, idx_pad):
    info = plsc.get_sparse_core_info()
    nc = info.num_cores

    mesh = plsc.VectorSubcoreMesh(core_axis_name="c", subcore_axis_name="s")

    @functools.partial(
        pl.kernel,
        out_type=jax.ShapeDtypeStruct((B, RUNW, DIM), jnp.float32),
        mesh=mesh,
        scratch_types=[
            pltpu.VMEM((RUNW,), jnp.int32),
            pltpu.VMEM((RUNW, DIM), jnp.float32),
            pltpu.SemaphoreType.DMA,
        ],
    )
    def gather_k(table_hbm, idx_hbm, out_hbm, idx_v, rows_v, sem):
        wid = lax.axis_index("s") * nc + lax.axis_index("c")
        pltpu.sync_copy(idx_hbm.at[wid], idx_v)
        pltpu.async_copy(table_hbm.at[idx_v], rows_v, sem).wait()
        pltpu.sync_copy(rows_v, out_hbm.at[wid])

    return gather_k(answer_embeddings, idx_pad)


def kernel(visual_feat, text_feat, answer_embeddings, in_proj_w, in_proj_b,
           out_proj_w, out_proj_b, fusion_w, fusion_b, ln_g, ln_b, sim_w,
           sim_b):
    fused_norm = _fused_query(visual_feat, text_feat, in_proj_w, in_proj_b,
                              out_proj_w, out_proj_b, fusion_w, fusion_b,
                              ln_g, ln_b, sim_w, sim_b)
    scores, idx = _sim_topk(fused_norm, answer_embeddings)
    idx_pad = jnp.concatenate(
        [idx, jnp.zeros((B, RUNW - K), jnp.int32)], axis=1)
    emb = _gather_sc(answer_embeddings, idx_pad)[:, :K, :]
    return scores, idx, emb


# speedup vs baseline: 2.0704x; 1.2260x over previous
import functools

import jax
import jax.numpy as jnp
from jax import lax
from jax.experimental import pallas as pl
from jax.experimental.pallas import tpu as pltpu
from jax.experimental.pallas import tpu_sc as plsc

B = 32
DIM = 768
K = 10
NUM_ANSWERS = 100000
RUNW = 16
BIGI = 2**30

R = 5000
NB = NUM_ANSWERS // R
RP = 5120
NGB = RP // 128
NG = NB * NGB
GSEL = 16


def _dot_t(x, w):
    return lax.dot_general(x.astype(jnp.bfloat16), w.astype(jnp.bfloat16),
                           (((1,), (1,)), ((), ())),
                           preferred_element_type=jnp.float32)


def _fused_query(visual_feat, text_feat, in_proj_w, in_proj_b, out_proj_w,
                 out_proj_b, fusion_w, fusion_b, ln_g, ln_b, sim_w, sim_b):
    wv = in_proj_w[2 * DIM:]
    bv = in_proj_b[2 * DIM:]
    v = _dot_t(visual_feat, wv) + bv
    attended = _dot_t(v, out_proj_w) + out_proj_b
    fused_in = jnp.concatenate([visual_feat, attended], axis=-1)
    fused = _dot_t(fused_in, fusion_w) + fusion_b
    mu = jnp.mean(fused, axis=-1, keepdims=True)
    var = jnp.mean((fused - mu) ** 2, axis=-1, keepdims=True)
    fused = (fused - mu) / jnp.sqrt(var + 1e-5) * ln_g + ln_b
    fused = jax.nn.gelu(fused, approximate=False)
    fused_proj = _dot_t(fused, sim_w) + sim_b
    n = jnp.linalg.norm(fused_proj, axis=-1, keepdims=True)
    return fused_proj / jnp.clip(n, 1e-12, None)


def _sim_body(fn_ref, a_ref, scores_ref, gid_ref, cm_ref):
    i = pl.program_id(0)

    a = a_ref[...]
    ones = jnp.ones((1, DIM), jnp.float32)
    ss = lax.dot_general(a * a, ones, (((1,), (1,)), ((), ())),
                         precision=lax.Precision.HIGHEST,
                         preferred_element_type=jnp.float32)
    an = a / jnp.maximum(jnp.sqrt(ss), 1e-12)
    s = _dot_t(fn_ref[...], an)
    scores_ref[0, :, :R] = s
    scores_ref[0, :, R:] = jnp.full((B, RP - R), -jnp.inf, jnp.float32)

    cm = jnp.concatenate(
        [jnp.max(s[:, g * 128:min((g + 1) * 128, R)], axis=1, keepdims=True)
         for g in range(NGB)], axis=1)
    cm_ref[i] = cm

    @pl.when(i == NB - 1)
    def _fin():
        c = cm_ref[...]
        gi = (lax.broadcasted_iota(jnp.int32, (NB, B, NGB), 0) * NGB
              + lax.broadcasted_iota(jnp.int32, (NB, B, NGB), 2))
        gl = []
        for _ in range(GSEL):
            m = jnp.max(jnp.max(c, axis=2), axis=0)
            hit = c == m[None, :, None]
            g = jnp.min(jnp.min(jnp.where(hit, gi, NG), axis=2), axis=0)
            gl.append(g[:, None])
            c = jnp.where(gi == g[None, :, None], -jnp.inf, c)
        gid_ref[...] = jnp.concatenate(gl, axis=1)


def _sim_groups(fused_norm, answer_embeddings):
    return pl.pallas_call(
        _sim_body,
        grid=(NB,),
        in_specs=[
            pl.BlockSpec((B, DIM), lambda i: (0, 0)),
            pl.BlockSpec((R, DIM), lambda i: (i, 0)),
        ],
        out_specs=[
            pl.BlockSpec((1, B, RP), lambda i: (i, 0, 0)),
            pl.BlockSpec((B, GSEL), lambda i: (0, 0)),
        ],
        out_shape=[
            jax.ShapeDtypeStruct((NB, B, RP), jnp.float32),
            jax.ShapeDtypeStruct((B, GSEL), jnp.int32),
        ],
        scratch_shapes=[
            pltpu.VMEM((NB, B, NGB), jnp.float32),
        ],
    )(fused_norm, answer_embeddings)


NCHK = 8


def _select_gather_sc(answer_embeddings, scores128, gids):
    info = plsc.get_sparse_core_info()
    nc = info.num_cores

    mesh = plsc.VectorSubcoreMesh(core_axis_name="c", subcore_axis_name="s")

    @functools.partial(
        pl.kernel,
        out_type=[
            jax.ShapeDtypeStruct((B, 16, DIM), jnp.float32),
            jax.ShapeDtypeStruct((B, 16), jnp.float32),
            jax.ShapeDtypeStruct((B, 16), jnp.int32),
        ],
        mesh=mesh,
        scratch_types=[
            pltpu.VMEM((GSEL,), jnp.int32),
            pltpu.VMEM((GSEL,), jnp.int32),
            pltpu.VMEM((GSEL, 128), jnp.float32),
            pltpu.VMEM((16,), jnp.int32),
            pltpu.VMEM((16, DIM), jnp.float32),
            pltpu.VMEM((16,), jnp.float32),
            pltpu.VMEM((16,), jnp.int32),
            pltpu.SemaphoreType.DMA,
        ],
    )
    def sel_k(table_hbm, s128_hbm, gid_hbm, emb_o, sc_o, si_o,
              gid_v, rowb, candb, eidx_v, rows_v, sv, iv, sem):
        q = lax.axis_index("s") * nc + lax.axis_index("c")
        pltpu.sync_copy(gid_hbm.at[q], gid_v)
        gid = gid_v[...]
        blk = lax.div(gid, jnp.full((16,), NGB, jnp.int32))
        rb = gid - blk * NGB
        rowb[...] = (blk * B + q) * NGB + rb
        pltpu.async_copy(s128_hbm.at[rowb], candb, sem).wait()

        li = lax.iota(jnp.int32, 16)
        nbig = jnp.full((16,), -BIGI, jnp.int32)

        def allmax(x):
            for step in (1, 2, 4, 8):
                x = jnp.maximum(x, jnp.take(x, jnp.bitwise_xor(li, step)))
            return x

        ebase = blk * R + rb * 128
        cands, eids = [], []
        for j in range(GSEL):
            esp = jnp.take(ebase, jnp.full((16,), j, jnp.int32))
            for c in range(NCHK):
                cands.append(candb[j, pl.ds(c * 16, 16)])
                eids.append(esp + (c * 16 + li))
        nv = len(cands)

        neg = jnp.full((16,), -jnp.inf, jnp.float32)
        bigv = jnp.full((16,), BIGI, jnp.int32)
        sv_vec = jnp.zeros((16,), jnp.float32)
        iv_vec = jnp.zeros((16,), jnp.int32)
        for r in range(K):
            f = cands[0]
            for m in range(1, nv):
                f = jnp.maximum(f, cands[m])
            msp = allmax(f)
            g = bigv
            for m in range(nv):
                g = jnp.minimum(g, jnp.where(cands[m] == msp, eids[m], bigv))
            ssp = -allmax(-g)
            for m in range(nv):
                cands[m] = jnp.where(eids[m] == ssp, neg, cands[m])
            sv_vec = jnp.where(li == r, msp, sv_vec)
            iv_vec = jnp.where(li == r, ssp, iv_vec)

        eidx_v[...] = jnp.where(li < K, iv_vec, 0)
        sv[...] = sv_vec
        iv[...] = iv_vec
        pltpu.async_copy(table_hbm.at[eidx_v], rows_v, sem).wait()
        pltpu.sync_copy(rows_v, emb_o.at[q])
        pltpu.sync_copy(sv, sc_o.at[q])
        pltpu.sync_copy(iv, si_o.at[q])

    return sel_k(answer_embeddings, scores128, gids)


def kernel(visual_feat, text_feat, answer_embeddings, in_proj_w, in_proj_b,
           out_proj_w, out_proj_b, fusion_w, fusion_b, ln_g, ln_b, sim_w,
           sim_b):
    fused_norm = _fused_query(visual_feat, text_feat, in_proj_w, in_proj_b,
                              out_proj_w, out_proj_b, fusion_w, fusion_b,
                              ln_g, ln_b, sim_w, sim_b)
    scores, gids = _sim_groups(fused_norm, answer_embeddings)
    scores128 = scores.reshape((NB * B * RP) // 128, 128)
    emb16, sc16, si16 = _select_gather_sc(answer_embeddings, scores128, gids)
    return sc16[:, :K], si16[:, :K], emb16[:, :K, :]
